# Initial kernel scaffold; baseline (speedup 1.0000x reference)
import functools

import jax
import jax.numpy as jnp
from jax import lax
from jax.experimental import pallas as pl
from jax.experimental.pallas import tpu as pltpu
from jax.experimental.pallas import tpu_sc as plsc

NC = 2
NS = 16
NW = NC * NS

CE = 1024
SUB = 128
CN = 128


def _seg_sum_sc(nodes, edges, nids, eids, z_e, z_n):
  N, DF = nodes.shape
  E, DE = edges.shape
  G = z_e.shape[0]
  GP = G // NS

  n_ec = E // CE
  e_rem = n_ec % NW
  e_base_cnt = n_ec // NW
  e_tail = E - n_ec * CE
  assert e_tail % SUB == 0 and e_tail // SUB <= CE // SUB

  n_nc = N // CN
  n_rem = n_nc % NW
  n_base_cnt = n_nc // NW
  n_tail = N - n_nc * CN
  assert n_tail % 8 == 0

  mesh = plsc.VectorSubcoreMesh(core_axis_name="c", subcore_axis_name="s")

  @functools.partial(
      pl.kernel,
      out_type=(
          jax.ShapeDtypeStruct((NC, G, DE), jnp.float32),
          jax.ShapeDtypeStruct((NC, G, DF), jnp.float32),
      ),
      mesh=mesh,
      scratch_types=[
          pltpu.VMEM((CE, DE), jnp.float32),
          pltpu.VMEM((SUB,), jnp.int32),
          pltpu.VMEM((CN, DF), jnp.float32),
          pltpu.VMEM((CN,), jnp.int32),
          pltpu.VMEM((max(n_tail, 8),), jnp.int32),
          pltpu.VMEM_SHARED((G, DE), jnp.float32),
          pltpu.VMEM_SHARED((G, DF), jnp.float32),
      ],
  )
  def k(nodes_h, edges_h, nids_h, eids_h, ze_h, zn_h, oute_h, outn_h,
        ebuf, eidx, nbuf, nidx, ntidx, acc_e, acc_n):
    c = lax.axis_index("c")
    s = lax.axis_index("s")
    w = c * NS + s

    pltpu.sync_copy(ze_h.at[pl.ds(s * GP, GP)], acc_e.at[pl.ds(s * GP, GP)])
    pltpu.sync_copy(zn_h.at[pl.ds(s * GP, GP)], acc_n.at[pl.ds(s * GP, GP)])
    plsc.subcore_barrier()

    def ebody(i, carry):
      base = (w + i * NW) * CE
      pltpu.sync_copy(edges_h.at[pl.ds(base, CE)], ebuf)
      for j in range(CE // SUB):
        pltpu.sync_copy(eids_h.at[pl.ds(base + j * SUB, SUB)], eidx)
        pltpu.sync_copy(ebuf.at[pl.ds(j * SUB, SUB)], acc_e.at[eidx],
                        add=True)
      return carry

    cnt_e = e_base_cnt + jnp.where(w < e_rem, 1, 0)
    lax.fori_loop(0, cnt_e, ebody, 0)

    if e_tail:
      @pl.when(w == e_rem)
      def _():
        base = n_ec * CE
        pltpu.sync_copy(edges_h.at[pl.ds(base, e_tail)],
                        ebuf.at[pl.ds(0, e_tail)])
        for j in range(e_tail // SUB):
          pltpu.sync_copy(eids_h.at[pl.ds(base + j * SUB, SUB)], eidx)
          pltpu.sync_copy(ebuf.at[pl.ds(j * SUB, SUB)], acc_e.at[eidx],
                          add=True)

    def nbody(i, carry):
      base = (w + i * NW) * CN
      pltpu.sync_copy(nodes_h.at[pl.ds(base, CN)], nbuf)
      pltpu.sync_copy(nids_h.at[pl.ds(base, CN)], nidx)
      pltpu.sync_copy(nbuf, acc_n.at[nidx], add=True)
      return carry

    cnt_n = n_base_cnt + jnp.where(w < n_rem, 1, 0)
    lax.fori_loop(0, cnt_n, nbody, 0)

    if n_tail:
      @pl.when(w == NW - 1)
      def _():
        base = n_nc * CN
        pltpu.sync_copy(nodes_h.at[pl.ds(base, n_tail)],
                        nbuf.at[pl.ds(0, n_tail)])
        pltpu.sync_copy(nids_h.at[pl.ds(base, n_tail)],
                        ntidx.at[pl.ds(0, n_tail)])
        pltpu.sync_copy(nbuf.at[pl.ds(0, n_tail)],
                        acc_n.at[ntidx.at[pl.ds(0, n_tail)]], add=True)

    plsc.subcore_barrier()

    pltpu.sync_copy(acc_e.at[pl.ds(s * GP, GP)],
                    oute_h.at[c, pl.ds(s * GP, GP)])
    pltpu.sync_copy(acc_n.at[pl.ds(s * GP, GP)],
                    outn_h.at[c, pl.ds(s * GP, GP)])

  return k(nodes, edges, nids, eids, z_e, z_n)


def _tc_body(agge_ref, aggn_ref, g_ref, w_ref, b_ref, out_ref):
  de = agge_ref.shape[2]
  df = aggn_ref.shape[2]
  acc_e = agge_ref[0] + agge_ref[1]
  acc_n = aggn_ref[0] + aggn_ref[1]
  out = jnp.dot(acc_e, w_ref[0:de, :], preferred_element_type=jnp.float32)
  out += jnp.dot(acc_n, w_ref[de:de + df, :],
                 preferred_element_type=jnp.float32)
  out += jnp.dot(g_ref[...], w_ref[de + df:, :],
                 preferred_element_type=jnp.float32)
  out_ref[...] = out + b_ref[...]


def kernel(nodes, edges, globals_, node_graph_ids, edge_graph_ids, W, b):
  G, DG = globals_.shape
  DE = edges.shape[1]
  nids = node_graph_ids.astype(jnp.int32)
  eids = edge_graph_ids.astype(jnp.int32)
  z_e = jnp.zeros((G, DE), jnp.float32)
  z_n = jnp.zeros((G, nodes.shape[1]), jnp.float32)

  agg_e, agg_n = _seg_sum_sc(nodes, edges, nids, eids, z_e, z_n)

  out = pl.pallas_call(
      _tc_body,
      out_shape=jax.ShapeDtypeStruct((G, W.shape[1]), jnp.float32),
  )(agg_e, agg_n, globals_, W, b.reshape(1, -1))
  return out


# speedup vs baseline: 5.0150x; 5.0150x over previous
import functools

import jax
import jax.numpy as jnp
from jax import lax
from jax.experimental import pallas as pl
from jax.experimental.pallas import tpu as pltpu
from jax.experimental.pallas import tpu_sc as plsc

NC = 2
NS = 16
NW = NC * NS

CE = 1024
SUB = 128
CN = 128


def _seg_sum_sc(nodes, edges, nids, eids, z_e, z_n):
  N, DF = nodes.shape
  E, DE = edges.shape
  G = z_e.shape[0]
  GP = G // NS

  n_ec = E // CE
  e_rem = n_ec % NW
  e_base_cnt = n_ec // NW
  e_tail = E - n_ec * CE
  assert e_tail % SUB == 0 and e_tail // SUB <= CE // SUB

  n_nc = N // CN
  n_rem = n_nc % NW
  n_base_cnt = n_nc // NW
  n_tail = N - n_nc * CN
  assert n_tail % 8 == 0

  mesh = plsc.VectorSubcoreMesh(core_axis_name="c", subcore_axis_name="s")

  @functools.partial(
      pl.kernel,
      out_type=(
          jax.ShapeDtypeStruct((NC, G, DE), jnp.float32),
          jax.ShapeDtypeStruct((NC, G, DF), jnp.float32),
      ),
      mesh=mesh,
      compiler_params=pltpu.CompilerParams(use_tc_tiling_on_sc=False),
      scratch_types=[
          pltpu.VMEM((CE, DE), jnp.float32),
          pltpu.VMEM((SUB,), jnp.int32),
          pltpu.VMEM((CN, DF), jnp.float32),
          pltpu.VMEM((CN,), jnp.int32),
          pltpu.VMEM((max(n_tail, 8),), jnp.int32),
          pltpu.VMEM_SHARED((G, DE), jnp.float32),
          pltpu.VMEM_SHARED((G, DF), jnp.float32),
      ],
  )
  def k(nodes_h, edges_h, nids_h, eids_h, ze_h, zn_h, oute_h, outn_h,
        ebuf, eidx, nbuf, nidx, ntidx, acc_e, acc_n):
    c = lax.axis_index("c")
    s = lax.axis_index("s")
    w = c * NS + s

    pltpu.sync_copy(ze_h.at[pl.ds(s * GP, GP)], acc_e.at[pl.ds(s * GP, GP)])
    pltpu.sync_copy(zn_h.at[pl.ds(s * GP, GP)], acc_n.at[pl.ds(s * GP, GP)])
    plsc.subcore_barrier()

    def ebody(i, carry):
      base = (w + i * NW) * CE
      pltpu.sync_copy(edges_h.at[pl.ds(base, CE)], ebuf)
      for j in range(CE // SUB):
        pltpu.sync_copy(eids_h.at[pl.ds(base + j * SUB, SUB)], eidx)
        pltpu.sync_copy(ebuf.at[pl.ds(j * SUB, SUB)], acc_e.at[eidx],
                        add=True)
      return carry

    cnt_e = e_base_cnt + jnp.where(w < e_rem, 1, 0)
    lax.fori_loop(0, cnt_e, ebody, 0)

    if e_tail:
      @pl.when(w == e_rem)
      def _():
        base = n_ec * CE
        pltpu.sync_copy(edges_h.at[pl.ds(base, e_tail)],
                        ebuf.at[pl.ds(0, e_tail)])
        for j in range(e_tail // SUB):
          pltpu.sync_copy(eids_h.at[pl.ds(base + j * SUB, SUB)], eidx)
          pltpu.sync_copy(ebuf.at[pl.ds(j * SUB, SUB)], acc_e.at[eidx],
                          add=True)

    def nbody(i, carry):
      base = (w + i * NW) * CN
      pltpu.sync_copy(nodes_h.at[pl.ds(base, CN)], nbuf)
      pltpu.sync_copy(nids_h.at[pl.ds(base, CN)], nidx)
      pltpu.sync_copy(nbuf, acc_n.at[nidx], add=True)
      return carry

    cnt_n = n_base_cnt + jnp.where(w < n_rem, 1, 0)
    lax.fori_loop(0, cnt_n, nbody, 0)

    if n_tail:
      @pl.when(w == NW - 1)
      def _():
        base = n_nc * CN
        pltpu.sync_copy(nodes_h.at[pl.ds(base, n_tail)],
                        nbuf.at[pl.ds(0, n_tail)])
        pltpu.sync_copy(nids_h.at[pl.ds(base, n_tail)],
                        ntidx.at[pl.ds(0, n_tail)])
        pltpu.sync_copy(nbuf.at[pl.ds(0, n_tail)],
                        acc_n.at[ntidx.at[pl.ds(0, n_tail)]], add=True)

    plsc.subcore_barrier()

    pltpu.sync_copy(acc_e.at[pl.ds(s * GP, GP)],
                    oute_h.at[c, pl.ds(s * GP, GP)])
    pltpu.sync_copy(acc_n.at[pl.ds(s * GP, GP)],
                    outn_h.at[c, pl.ds(s * GP, GP)])

  return k(nodes, edges, nids, eids, z_e, z_n)


def _tc_body(agge_ref, aggn_ref, g_ref, w_ref, b_ref, out_ref):
  de = agge_ref.shape[2]
  df = aggn_ref.shape[2]
  acc_e = agge_ref[0] + agge_ref[1]
  acc_n = aggn_ref[0] + aggn_ref[1]
  out = jnp.dot(acc_e, w_ref[0:de, :], preferred_element_type=jnp.float32)
  out += jnp.dot(acc_n, w_ref[de:de + df, :],
                 preferred_element_type=jnp.float32)
  out += jnp.dot(g_ref[...], w_ref[de + df:, :],
                 preferred_element_type=jnp.float32)
  out_ref[...] = out + b_ref[...]


def kernel(nodes, edges, globals_, node_graph_ids, edge_graph_ids, W, b):
  G, DG = globals_.shape
  DE = edges.shape[1]
  nids = node_graph_ids.astype(jnp.int32)
  eids = edge_graph_ids.astype(jnp.int32)
  z_e = jnp.zeros((G, DE), jnp.float32)
  z_n = jnp.zeros((G, nodes.shape[1]), jnp.float32)

  agg_e, agg_n = _seg_sum_sc(nodes, edges, nids, eids, z_e, z_n)

  out = pl.pallas_call(
      _tc_body,
      out_shape=jax.ShapeDtypeStruct((G, W.shape[1]), jnp.float32),
  )(agg_e, agg_n, globals_, W, b.reshape(1, -1))
  return out


# speedup vs baseline: 5.9692x; 1.1903x over previous
import functools

import jax
import jax.numpy as jnp
from jax import lax
from jax.experimental import pallas as pl
from jax.experimental.pallas import tpu as pltpu
from jax.experimental.pallas import tpu_sc as plsc

NC = 2
NS = 16
NW = NC * NS

CEP = 128
CN = 128
L = 16


def _sc_nodes(nodes, nids, z_n):
  N, DF = nodes.shape
  G = z_n.shape[0]
  GP = G // NS

  n_nc = N // CN
  n_rem = n_nc % NW
  n_base_cnt = n_nc // NW
  n_tail = N - n_nc * CN
  assert n_tail % 8 == 0

  mesh = plsc.VectorSubcoreMesh(core_axis_name="c", subcore_axis_name="s")

  @functools.partial(
      pl.kernel,
      out_type=jax.ShapeDtypeStruct((NC, G, DF), jnp.float32),
      mesh=mesh,
      compiler_params=pltpu.CompilerParams(use_tc_tiling_on_sc=False, needs_layout_passes=False),
      scratch_types=[
          pltpu.VMEM((CN, DF), jnp.float32),
          pltpu.VMEM((CN,), jnp.int32),
          pltpu.VMEM((max(n_tail, 8),), jnp.int32),
          pltpu.VMEM_SHARED((G, DF), jnp.float32),
      ],
  )
  def k(nodes_h, nids_h, zn_h, outn_h, nbuf, nidx, ntidx, acc_n):
    c = lax.axis_index("c")
    s = lax.axis_index("s")
    w = c * NS + s

    pltpu.sync_copy(zn_h.at[pl.ds(s * GP, GP)], acc_n.at[pl.ds(s * GP, GP)])
    plsc.subcore_barrier()

    def nbody(i, carry):
      base = (w + i * NW) * CN
      pltpu.sync_copy(nodes_h.at[pl.ds(base, CN)], nbuf)
      pltpu.sync_copy(nids_h.at[pl.ds(base, CN)], nidx)
      pltpu.sync_copy(nbuf, acc_n.at[nidx], add=True)
      return carry

    cnt_n = n_base_cnt + jnp.where(w < n_rem, 1, 0)
    lax.fori_loop(0, cnt_n, nbody, 0)

    if n_tail:
      @pl.when(w == NW - 1)
      def _():
        base = n_nc * CN
        pltpu.sync_copy(nodes_h.at[pl.ds(base, n_tail)],
                        nbuf.at[pl.ds(0, n_tail)])
        pltpu.sync_copy(nids_h.at[pl.ds(base, n_tail)],
                        ntidx.at[pl.ds(0, n_tail)])
        pltpu.sync_copy(nbuf.at[pl.ds(0, n_tail)],
                        acc_n.at[ntidx.at[pl.ds(0, n_tail)]], add=True)

    plsc.subcore_barrier()
    pltpu.sync_copy(acc_n.at[pl.ds(s * GP, GP)],
                    outn_h.at[c, pl.ds(s * GP, GP)])

  return k(nodes, nids, z_n)


def _sc_edges(e128, eids, z_ep, z_ec, DE):
  R = e128.shape[0]
  EW = 128 // DE
  E = R * EW
  G = z_ec.shape[0]
  GP = G // NS
  KV = CEP // L

  n_ec = R // CEP
  e_rem = n_ec % NW
  e_base_cnt = n_ec // NW
  t_rows = R - n_ec * CEP
  assert t_rows % L == 0

  mesh = plsc.VectorSubcoreMesh(core_axis_name="c", subcore_axis_name="s")

  @functools.partial(
      pl.kernel,
      out_type=(
          jax.ShapeDtypeStruct((NC, G, 128), jnp.float32),
          jax.ShapeDtypeStruct((NC, G, DE), jnp.float32),
      ),
      mesh=mesh,
      compiler_params=pltpu.CompilerParams(use_tc_tiling_on_sc=False, needs_layout_passes=False),
      scratch_types=[
          pltpu.VMEM((CEP, 128), jnp.float32),
          pltpu.VMEM((CEP * EW,), jnp.int32),
          pltpu.VMEM((CEP,), jnp.int32),
          pltpu.VMEM((max(t_rows, 8),), jnp.int32),
          pltpu.VMEM((L, DE), jnp.float32),
          pltpu.VMEM((L,), jnp.int32),
          pltpu.VMEM_SHARED((G + 8, 128), jnp.float32),
          pltpu.VMEM_SHARED((G + 8, DE), jnp.float32),
      ],
  )
  def k(e128_h, eids_h, zep_h, zec_h, outp_h, outc_h,
        ebuf, idbuf, qbuf, qtail, patch, idxm, acc_ep, acc_ec):
    c = lax.axis_index("c")
    s = lax.axis_index("s")
    w = c * NS + s
    iota = lax.iota(jnp.int32, L)
    lomask = iota < EW
    for j in range(EW, L):
      patch[j, :] = jnp.zeros((DE,), jnp.float32)

    pltpu.sync_copy(zep_h.at[pl.ds(s * GP, GP)],
                    acc_ep.at[pl.ds(s * GP, GP)])
    pltpu.sync_copy(zec_h.at[pl.ds(s * GP, GP)],
                    acc_ec.at[pl.ds(s * GP, GP)])
    plsc.subcore_barrier()

    def fix_mixed(m, kbase):
      def cond(carry):
        return jnp.any(carry)

      def body(carry):
        m = carry
        lvec = plsc.all_reduce_ffs(m)
        l = jnp.max(lvec)
        p = kbase + l
        idv = plsc.load_gather(idbuf.at[:], [p * EW + iota], mask=lomask)
        idxm[...] = jnp.where(lomask, idv, G)
        for j in range(EW):
          patch[j, :] = ebuf[p, pl.ds(j * DE, DE)]
        pltpu.sync_copy(patch, acc_ec.at[idxm], add=True)
        return m & (iota != l)

      lax.while_loop(cond, body, m)

    def process_chunk(rows, qref, rowbase):
      for kk in range(rows // L):
        fidx = (kk * L + iota) * EW
        first = plsc.load_gather(idbuf.at[:], [fidx])
        last = plsc.load_gather(idbuf.at[:], [fidx + (EW - 1)])
        m = first != last
        qref[pl.ds(kk * L, L)] = jnp.where(m, G, first)
        @pl.when(jnp.any(m))
        def _():
          fix_mixed(m, kk * L)

    def ebody(i, carry):
      rbase = (w + i * NW) * CEP
      pltpu.sync_copy(e128_h.at[pl.ds(rbase, CEP)], ebuf)
      pltpu.sync_copy(eids_h.at[pl.ds(rbase * EW, CEP * EW)], idbuf)
      process_chunk(CEP, qbuf, rbase)
      pltpu.sync_copy(ebuf, acc_ep.at[qbuf], add=True)
      return carry

    cnt_e = e_base_cnt + jnp.where(w < e_rem, 1, 0)
    lax.fori_loop(0, cnt_e, ebody, 0)

    if t_rows:
      @pl.when(w == e_rem)
      def _():
        rbase = n_ec * CEP
        pltpu.sync_copy(e128_h.at[pl.ds(rbase, t_rows)],
                        ebuf.at[pl.ds(0, t_rows)])
        pltpu.sync_copy(eids_h.at[pl.ds(rbase * EW, t_rows * EW)],
                        idbuf.at[pl.ds(0, t_rows * EW)])
        process_chunk(t_rows, qtail, rbase)
        pltpu.sync_copy(ebuf.at[pl.ds(0, t_rows)], acc_ep.at[qtail],
                        add=True)

    plsc.subcore_barrier()
    pltpu.sync_copy(acc_ep.at[pl.ds(s * GP, GP)],
                    outp_h.at[c, pl.ds(s * GP, GP)])
    pltpu.sync_copy(acc_ec.at[pl.ds(s * GP, GP)],
                    outc_h.at[c, pl.ds(s * GP, GP)])

  return k(e128, eids, z_ep, z_ec)


def _tc_body(aggp_ref, aggc_ref, aggn_ref, g_ref, w_ref, b_ref, out_ref):
  de = aggc_ref.shape[2]
  df = aggn_ref.shape[2]
  acc_p = aggp_ref[0] + aggp_ref[1]
  acc_c = aggc_ref[0] + aggc_ref[1]
  acc_n = aggn_ref[0] + aggn_ref[1]
  folded = acc_c
  for j in range(128 // de):
    folded = folded + acc_p[:, j * de:(j + 1) * de]
  out = jnp.dot(folded, w_ref[0:de, :], preferred_element_type=jnp.float32)
  out += jnp.dot(acc_n, w_ref[de:de + df, :],
                 preferred_element_type=jnp.float32)
  out += jnp.dot(g_ref[...], w_ref[de + df:, :],
                 preferred_element_type=jnp.float32)
  out_ref[...] = out + b_ref[...]


def kernel(nodes, edges, globals_, node_graph_ids, edge_graph_ids, W, b):
  G, DG = globals_.shape
  E, DE = edges.shape
  DF = nodes.shape[1]
  nids = node_graph_ids.astype(jnp.int32)
  eids = edge_graph_ids.astype(jnp.int32)
  e128 = edges.reshape(E * DE // 128, 128)
  z_n = jnp.zeros((G, DF), jnp.float32)
  z_ep = jnp.zeros((G, 128), jnp.float32)
  z_ec = jnp.zeros((G, DE), jnp.float32)
  agg_n = _sc_nodes(nodes, nids, z_n)
  agg_p, agg_c = _sc_edges(e128, eids, z_ep, z_ec, DE)

  out = pl.pallas_call(
      _tc_body,
      out_shape=jax.ShapeDtypeStruct((G, W.shape[1]), jnp.float32),
  )(agg_p, agg_c, agg_n, globals_, W, b.reshape(1, -1))
  return out
